# baseline (speedup 1.0000x reference)
import jax
import jax.numpy as jnp
from jax import lax
from jax.experimental import pallas as pl
from jax.experimental.pallas import tpu as pltpu
from jax.experimental.pallas import tpu_sc as plsc

N = 10000
NP = 10240
E = 160000
EP = 163840
D = 256
H = 128

CH = 128
NROW = EP // CH
RPT = NP // 16
SROW = NROW // 16
HROW = SROW // 2
DROW = NROW // 32
NBUF = 2

_f32 = jnp.float32


def _mesh():
    return plsc.VectorSubcoreMesh(core_axis_name="c", subcore_axis_name="s",
                                  num_cores=2, num_subcores=16)


def _deg_body(dst_hbm, zeros_hbm, ones_hbm, out_hbm, acc, idx_v, ones_v,
              *sems):
    c = lax.axis_index("c")
    s = lax.axis_index("s")
    wid = c * 16 + s
    rows = pl.ds(s * RPT, RPT)
    pltpu.sync_copy(dst_hbm.at[pl.ds(wid * DROW, DROW)], idx_v)
    pltpu.sync_copy(ones_hbm, ones_v)
    pltpu.sync_copy(zeros_hbm, acc.at[rows])
    plsc.subcore_barrier()
    for g in range(DROW // NBUF):
        descs = []
        for b in range(NBUF):
            k = g * NBUF + b
            descs.append(pltpu.async_copy(
                ones_v, acc.at[idx_v.at[k]], sems[b], add=True))
        for d in descs:
            d.wait()
    plsc.subcore_barrier()
    pltpu.sync_copy(acc.at[rows], out_hbm.at[c].at[rows])


def _sc_degree(dst2d):
    zeros = jnp.zeros((RPT, H), _f32)
    ones = jnp.ones((CH, H), _f32)
    return pl.kernel(
        _deg_body,
        out_type=jax.ShapeDtypeStruct((2, NP, H), _f32),
        mesh=_mesh(),
        scratch_types=[
            pltpu.VMEM_SHARED((NP, H), _f32),
            pltpu.VMEM((DROW, CH), jnp.int32),
            pltpu.VMEM((CH, H), _f32),
        ] + [pltpu.SemaphoreType.DMA] * NBUF,
    )(dst2d, zeros, ones)


def _scatter_body(ylo, yhi, src_hbm, dst_hbm, zeros_hbm, zlo_out, zhi_out,
                  zacc, src_v, dst_v, *bufs_sems):
    rows_v = bufs_sems[:NBUF]
    gsem = bufs_sems[NBUF:2 * NBUF]
    asem = bufs_sems[2 * NBUF:3 * NBUF]
    c = lax.axis_index("c")
    s = lax.axis_index("s")
    pltpu.sync_copy(zeros_hbm, zacc.at[pl.ds(s * RPT, RPT)])
    plsc.subcore_barrier()

    def run(y_hbm):
        def gather(k, b):
            pltpu.async_copy(y_hbm.at[src_v.at[k]], rows_v[b], gsem[b])

        def gather_wait(k, b):
            pltpu.make_async_copy(y_hbm.at[src_v.at[k]], rows_v[b],
                                  gsem[b]).wait()

        def add(k, b):
            return pltpu.async_copy(rows_v[b], zacc.at[dst_v.at[k]], asem[b],
                                    add=True)

        for ph in range(SROW // HROW):
            irows = pl.ds(s * SROW + ph * HROW, HROW)
            pltpu.sync_copy(src_hbm.at[irows], src_v)
            pltpu.sync_copy(dst_hbm.at[irows], dst_v)
            for b in range(NBUF):
                gather(b, b)

            def body(i, carry):
                k0 = i * NBUF
                adds = []
                for b in range(NBUF):
                    gather_wait(k0 + b, b)
                    adds.append(add(k0 + b, b))
                for b in range(NBUF):
                    adds[b].wait()

                    @pl.when(k0 + NBUF + b < HROW)
                    def _():
                        gather(k0 + NBUF + b, b)
                return carry

            lax.fori_loop(0, HROW // NBUF, body, 0)

    pl.when(c == 0)(lambda: run(ylo))
    pl.when(c != 0)(lambda: run(yhi))
    plsc.subcore_barrier()
    orow = pl.ds(s * RPT, RPT)
    pl.when(c == 0)(lambda: pltpu.sync_copy(zacc.at[orow], zlo_out.at[orow]))
    pl.when(c != 0)(lambda: pltpu.sync_copy(zacc.at[orow], zhi_out.at[orow]))


def _sc_scatter(ylo, yhi, src2d, dst2d):
    zeros = jnp.zeros((RPT, H), _f32)
    return pl.kernel(
        _scatter_body,
        out_type=[jax.ShapeDtypeStruct((NP, H), _f32),
                  jax.ShapeDtypeStruct((NP, H), _f32)],
        mesh=_mesh(),
        scratch_types=[
            pltpu.VMEM_SHARED((NP, H), _f32),
            pltpu.VMEM((HROW, CH), jnp.int32),
            pltpu.VMEM((HROW, CH), jnp.int32),
        ] + [pltpu.VMEM((CH, H), _f32)] * NBUF
          + [pltpu.SemaphoreType.DMA] * (2 * NBUF),
    )(ylo, yhi, src2d, dst2d, zeros)


BR = 400
GRID = N // BR


def _dinv_from(dc_blk):
    deg = dc_blk[0, :, 0] + dc_blk[1, :, 0] + 1.0
    return lax.rsqrt(jnp.maximum(deg, 1e-12))


def _mm1_body(x_ref, w_ref, dc_ref, ylo_ref, yhi_ref):
    xw = jnp.dot(x_ref[...], w_ref[...], preferred_element_type=_f32)
    dinv = _dinv_from(dc_ref[...])
    y = xw * dinv[:, None]
    ylo_ref[...] = y[:, :H]
    yhi_ref[...] = y[:, H:]


def _tc_mm1(x, W1, dcount):
    return pl.pallas_call(
        _mm1_body,
        grid=(GRID,),
        in_specs=[
            pl.BlockSpec((BR, D), lambda i: (i, 0)),
            pl.BlockSpec((D, D), lambda i: (0, 0)),
            pl.BlockSpec((2, BR, H), lambda i: (0, i, 0)),
        ],
        out_specs=[pl.BlockSpec((BR, H), lambda i: (i, 0)),
                   pl.BlockSpec((BR, H), lambda i: (i, 0))],
        out_shape=[jax.ShapeDtypeStruct((N, H), _f32),
                   jax.ShapeDtypeStruct((N, H), _f32)],
    )(x, W1, dcount)


def _mm2_body(zlo_ref, zhi_ref, ylo_ref, yhi_ref, dc_ref, b_ref, w_ref,
              olo_ref, ohi_ref):
    z = jnp.concatenate([zlo_ref[...], zhi_ref[...]], axis=1)
    y = jnp.concatenate([ylo_ref[...], yhi_ref[...]], axis=1)
    dinv = _dinv_from(dc_ref[...])
    h = jnp.maximum(dinv[:, None] * (z + y) + b_ref[...][None, :], 0.0)
    xw = jnp.dot(h, w_ref[...], preferred_element_type=_f32)
    y2 = xw * dinv[:, None]
    olo_ref[...] = y2[:, :H]
    ohi_ref[...] = y2[:, H:]


def _tc_mm2(zlo, zhi, ylo, yhi, dcount, b1, W2):
    return pl.pallas_call(
        _mm2_body,
        grid=(GRID,),
        in_specs=[
            pl.BlockSpec((BR, H), lambda i: (i, 0)),
            pl.BlockSpec((BR, H), lambda i: (i, 0)),
            pl.BlockSpec((BR, H), lambda i: (i, 0)),
            pl.BlockSpec((BR, H), lambda i: (i, 0)),
            pl.BlockSpec((2, BR, H), lambda i: (0, i, 0)),
            pl.BlockSpec((D,), lambda i: (0,)),
            pl.BlockSpec((D, D), lambda i: (0, 0)),
        ],
        out_specs=[pl.BlockSpec((BR, H), lambda i: (i, 0)),
                   pl.BlockSpec((BR, H), lambda i: (i, 0))],
        out_shape=[jax.ShapeDtypeStruct((N, H), _f32),
                   jax.ShapeDtypeStruct((N, H), _f32)],
    )(zlo, zhi, ylo, yhi, dcount, b1, W2)


def _mm3_body(zlo_ref, zhi_ref, ylo_ref, yhi_ref, dc_ref, b_ref, wc_ref,
              bc_ref, out_ref):
    z = jnp.concatenate([zlo_ref[...], zhi_ref[...]], axis=1)
    y = jnp.concatenate([ylo_ref[...], yhi_ref[...]], axis=1)
    dinv = _dinv_from(dc_ref[...])
    h = jnp.maximum(dinv[:, None] * (z + y) + b_ref[...][None, :], 0.0)
    out_ref[...] = (jnp.dot(h, wc_ref[...], preferred_element_type=_f32)
                    + bc_ref[...][None, :])


def _tc_mm3(zlo, zhi, ylo, yhi, dcount, b2, Wc, bc):
    return pl.pallas_call(
        _mm3_body,
        grid=(GRID,),
        in_specs=[
            pl.BlockSpec((BR, H), lambda i: (i, 0)),
            pl.BlockSpec((BR, H), lambda i: (i, 0)),
            pl.BlockSpec((BR, H), lambda i: (i, 0)),
            pl.BlockSpec((BR, H), lambda i: (i, 0)),
            pl.BlockSpec((2, BR, H), lambda i: (0, i, 0)),
            pl.BlockSpec((D,), lambda i: (0,)),
            pl.BlockSpec((D, 2), lambda i: (0, 0)),
            pl.BlockSpec((2,), lambda i: (0,)),
        ],
        out_specs=pl.BlockSpec((BR, 2), lambda i: (i, 0)),
        out_shape=jax.ShapeDtypeStruct((N, 2), _f32),
    )(zlo, zhi, ylo, yhi, dcount, b2, Wc, bc)


def kernel(x, edge_index, W1, b1, W2, b2, Wc, bc):
    src = edge_index[0].astype(jnp.int32)
    dst = edge_index[1].astype(jnp.int32)
    npad = EP - E
    src2d = jnp.concatenate(
        [src, jnp.zeros((npad,), jnp.int32)]).reshape(NROW, CH)
    dst2d = jnp.concatenate(
        [dst, N + (jnp.arange(npad, dtype=jnp.int32) % (NP - N))]
    ).reshape(NROW, CH)
    dcount = _sc_degree(dst2d)
    y1lo, y1hi = _tc_mm1(x, W1, dcount)
    z1lo, z1hi = _sc_scatter(y1lo, y1hi, src2d, dst2d)
    y2lo, y2hi = _tc_mm2(z1lo, z1hi, y1lo, y1hi, dcount, b1, W2)
    z2lo, z2hi = _sc_scatter(y2lo, y2hi, src2d, dst2d)
    return _tc_mm3(z2lo, z2hi, y2lo, y2hi, dcount, b2, Wc, bc)


# speedup vs baseline: 1.0560x; 1.0560x over previous
import jax
import jax.numpy as jnp
from jax import lax
from jax.experimental import pallas as pl
from jax.experimental.pallas import tpu as pltpu
from jax.experimental.pallas import tpu_sc as plsc

N = 10000
NP = 10240
E = 160000
EP = 163840
D = 256
H = 128

CH = 128
NROW = EP // CH
RPT = NP // 16
SROW = NROW // 16
HROW = 40
DROW = NROW // 32
NBUF = 2

_f32 = jnp.float32


def _mesh():
    return plsc.VectorSubcoreMesh(core_axis_name="c", subcore_axis_name="s",
                                  num_cores=2, num_subcores=16)


def _deg_body(dst_hbm, zeros_hbm, ones_hbm, out_hbm, acc, idx_v, ones_v,
              *sems):
    c = lax.axis_index("c")
    s = lax.axis_index("s")
    wid = c * 16 + s
    rows = pl.ds(s * RPT, RPT)
    pltpu.sync_copy(dst_hbm.at[pl.ds(wid * DROW, DROW)], idx_v)
    pltpu.sync_copy(ones_hbm, ones_v)
    pltpu.sync_copy(zeros_hbm, acc.at[rows])
    plsc.subcore_barrier()
    for g in range(DROW // NBUF):
        descs = []
        for b in range(NBUF):
            k = g * NBUF + b
            descs.append(pltpu.async_copy(
                ones_v, acc.at[idx_v.at[k]], sems[b], add=True))
        for d in descs:
            d.wait()
    plsc.subcore_barrier()
    pltpu.sync_copy(acc.at[rows], out_hbm.at[c].at[rows])


def _sc_degree(dst2d):
    zeros = jnp.zeros((RPT, H), _f32)
    ones = jnp.ones((CH, H), _f32)
    return pl.kernel(
        _deg_body,
        out_type=jax.ShapeDtypeStruct((2, NP, H), _f32),
        mesh=_mesh(),
        scratch_types=[
            pltpu.VMEM_SHARED((NP, H), _f32),
            pltpu.VMEM((DROW, CH), jnp.int32),
            pltpu.VMEM((CH, H), _f32),
        ] + [pltpu.SemaphoreType.DMA] * NBUF,
    )(dst2d, zeros, ones)


def _scatter_body(y_hbm, src_hbm, dst_hbm, zeros_hbm, zlo_out, zhi_out,
                  zacc, src_v, dst_v, *bufs_sems):
    rows_v = bufs_sems[:NBUF]
    gsem = bufs_sems[NBUF:2 * NBUF]
    asem = bufs_sems[2 * NBUF:3 * NBUF]
    c = lax.axis_index("c")
    s = lax.axis_index("s")
    pltpu.sync_copy(zeros_hbm, zacc.at[pl.ds(s * RPT, RPT)])
    plsc.subcore_barrier()

    def run(off):
        def gather(k, b):
            pltpu.async_copy(y_hbm.at[src_v.at[k], pl.ds(off, H)],
                             rows_v[b], gsem[b])

        def gather_wait(k, b):
            pltpu.make_async_copy(y_hbm.at[src_v.at[k], pl.ds(off, H)],
                                  rows_v[b], gsem[b]).wait()

        def add(k, b):
            return pltpu.async_copy(rows_v[b], zacc.at[dst_v.at[k]], asem[b],
                                    add=True)

        for ph in range(SROW // HROW):
            irows = pl.ds(s * SROW + ph * HROW, HROW)
            pltpu.sync_copy(src_hbm.at[irows], src_v)
            pltpu.sync_copy(dst_hbm.at[irows], dst_v)
            for b in range(NBUF):
                gather(b, b)

            def body(i, carry):
                k0 = i * NBUF
                adds = []
                adds = []
                for b in range(NBUF):
                    gather_wait(k0 + b, b)
                    adds.append(add(k0 + b, b))
                for b in range(NBUF):
                    adds[b].wait()

                    @pl.when(k0 + NBUF + b < HROW)
                    def _():
                        gather(k0 + NBUF + b, b)
                return carry

            lax.fori_loop(0, HROW // NBUF, body, 0)

    pl.when(c == 0)(lambda: run(0))
    pl.when(c != 0)(lambda: run(H))
    plsc.subcore_barrier()
    orow = pl.ds(s * RPT, RPT)
    pl.when(c == 0)(lambda: pltpu.sync_copy(zacc.at[orow], zlo_out.at[orow]))
    pl.when(c != 0)(lambda: pltpu.sync_copy(zacc.at[orow], zhi_out.at[orow]))


def _sc_scatter(y, src2d, dst2d):
    zeros = jnp.zeros((RPT, H), _f32)
    return pl.kernel(
        _scatter_body,
        out_type=[jax.ShapeDtypeStruct((NP, H), _f32),
                  jax.ShapeDtypeStruct((NP, H), _f32)],
        mesh=_mesh(),
        scratch_types=[
            pltpu.VMEM_SHARED((NP, H), _f32),
            pltpu.VMEM((HROW, CH), jnp.int32),
            pltpu.VMEM((HROW, CH), jnp.int32),
        ] + [pltpu.VMEM((CH, H), _f32)] * NBUF
          + [pltpu.SemaphoreType.DMA] * (2 * NBUF),
    )(y, src2d, dst2d, zeros)


BR = 400
GRID = N // BR


def _dinv_from(dc_blk):
    deg = dc_blk[0, :, 0] + dc_blk[1, :, 0] + 1.0
    return lax.rsqrt(jnp.maximum(deg, 1e-12))


def _mm1_body(x_ref, w_ref, dc_ref, y_ref):
    xw = jnp.dot(x_ref[...], w_ref[...], preferred_element_type=_f32)
    dinv = _dinv_from(dc_ref[...])
    y_ref[...] = xw * dinv[:, None]


def _tc_mm1(x, W1, dcount):
    return pl.pallas_call(
        _mm1_body,
        grid=(GRID,),
        in_specs=[
            pl.BlockSpec((BR, D), lambda i: (i, 0)),
            pl.BlockSpec((D, D), lambda i: (0, 0)),
            pl.BlockSpec((2, BR, H), lambda i: (0, i, 0)),
        ],
        out_specs=pl.BlockSpec((BR, D), lambda i: (i, 0)),
        out_shape=jax.ShapeDtypeStruct((N, D), _f32),
    )(x, W1, dcount)


def _mm2_body(zlo_ref, zhi_ref, y_ref, dc_ref, b_ref, w_ref, o_ref):
    z = jnp.concatenate([zlo_ref[...], zhi_ref[...]], axis=1)
    dinv = _dinv_from(dc_ref[...])
    h = jnp.maximum(dinv[:, None] * (z + y_ref[...]) + b_ref[...][None, :],
                    0.0)
    xw = jnp.dot(h, w_ref[...], preferred_element_type=_f32)
    o_ref[...] = xw * dinv[:, None]


def _tc_mm2(zlo, zhi, y, dcount, b1, W2):
    return pl.pallas_call(
        _mm2_body,
        grid=(GRID,),
        in_specs=[
            pl.BlockSpec((BR, H), lambda i: (i, 0)),
            pl.BlockSpec((BR, H), lambda i: (i, 0)),
            pl.BlockSpec((BR, D), lambda i: (i, 0)),
            pl.BlockSpec((2, BR, H), lambda i: (0, i, 0)),
            pl.BlockSpec((D,), lambda i: (0,)),
            pl.BlockSpec((D, D), lambda i: (0, 0)),
        ],
        out_specs=pl.BlockSpec((BR, D), lambda i: (i, 0)),
        out_shape=jax.ShapeDtypeStruct((N, D), _f32),
    )(zlo, zhi, y, dcount, b1, W2)


def _mm3_body(zlo_ref, zhi_ref, y_ref, dc_ref, b_ref, wc_ref,
              bc_ref, out_ref):
    z = jnp.concatenate([zlo_ref[...], zhi_ref[...]], axis=1)
    dinv = _dinv_from(dc_ref[...])
    h = jnp.maximum(dinv[:, None] * (z + y_ref[...]) + b_ref[...][None, :],
                    0.0)
    out_ref[...] = (jnp.dot(h, wc_ref[...], preferred_element_type=_f32)
                    + bc_ref[...][None, :])


def _tc_mm3(zlo, zhi, y, dcount, b2, Wc, bc):
    return pl.pallas_call(
        _mm3_body,
        grid=(GRID,),
        in_specs=[
            pl.BlockSpec((BR, H), lambda i: (i, 0)),
            pl.BlockSpec((BR, H), lambda i: (i, 0)),
            pl.BlockSpec((BR, D), lambda i: (i, 0)),
            pl.BlockSpec((2, BR, H), lambda i: (0, i, 0)),
            pl.BlockSpec((D,), lambda i: (0,)),
            pl.BlockSpec((D, 2), lambda i: (0, 0)),
            pl.BlockSpec((2,), lambda i: (0,)),
        ],
        out_specs=pl.BlockSpec((BR, 2), lambda i: (i, 0)),
        out_shape=jax.ShapeDtypeStruct((N, 2), _f32),
    )(zlo, zhi, y, dcount, b2, Wc, bc)


def kernel(x, edge_index, W1, b1, W2, b2, Wc, bc):
    src = edge_index[0].astype(jnp.int32)
    dst = edge_index[1].astype(jnp.int32)
    npad = EP - E
    src2d = jnp.concatenate(
        [src, jnp.zeros((npad,), jnp.int32)]).reshape(NROW, CH)
    dst2d = jnp.concatenate(
        [dst, N + (jnp.arange(npad, dtype=jnp.int32) % (NP - N))]
    ).reshape(NROW, CH)
    dcount = _sc_degree(dst2d)
    y1 = _tc_mm1(x, W1, dcount)
    z1lo, z1hi = _sc_scatter(y1, src2d, dst2d)
    y2 = _tc_mm2(z1lo, z1hi, y1, dcount, b1, W2)
    z2lo, z2hi = _sc_scatter(y2, src2d, dst2d)
    return _tc_mm3(z2lo, z2hi, y2, dcount, b2, Wc, bc)
